# baseline (speedup 1.0000x reference)
import jax
import jax.numpy as jnp
from jax import lax
from jax.experimental import pallas as pl
from jax.experimental.pallas import tpu as pltpu
from jax.experimental.pallas import tpu_sc as plsc

N = 10000
E = 320000
D = 128
NC = 2
NS = 16
NW = NC * NS
CHUNK = 128
NCHUNKS = E // CHUNK
CPW = 80
EPAD = NW * CPW * CHUNK
R = 10112
RPS = R // NS
RLAST = N - 15 * RPS
assert NCHUNKS % 2 == 0 and CPW % 4 == 0 and RLAST % 8 == 0

_MESH = dict(core_axis_name="c", subcore_axis_name="s",
             num_cores=NC, num_subcores=NS)
_NOLAYOUT = pltpu.CompilerParams(needs_layout_passes=False)


def _make_sc_scatter():
    def body(y_hbm, ev_hbm, z_hbm, acc_out,
             eb, gb0, gb1, acc_sh, sem0, sem1):
        c = lax.axis_index("c")
        s = lax.axis_index("s")
        w = s * NC + c
        base = s * RPS
        nrows = jnp.where(s < NS - 1, RPS, RLAST)

        def _zinit(i, _):
            pltpu.sync_copy(z_hbm, acc_sh.at[pl.ds(base + i * 8, 8)])
            return 0
        lax.fori_loop(0, nrows // 8, _zinit, 0)
        plsc.subcore_barrier()

        nreal = jnp.clip(NCHUNKS - w * CPW, 0, CPW)

        def _pair(i, _):
            j0 = 2 * i
            g = j0 // 4
            r = 2 * (j0 % 4)

            def _load_group():
                pltpu.sync_copy(ev_hbm.at[w, pl.ds(g * 8, 8)], eb)
            pl.when(j0 % 4 == 0)(_load_group)

            pltpu.async_copy(y_hbm.at[eb.at[r]], gb0, sem0)
            pltpu.async_copy(y_hbm.at[eb.at[r + 2]], gb1, sem1)
            pltpu.make_async_copy(y_hbm.at[eb.at[r]], gb0, sem0).wait()
            pltpu.sync_copy(gb0, acc_sh.at[eb.at[r + 1]], add=True)
            pltpu.make_async_copy(y_hbm.at[eb.at[r + 2]], gb1, sem1).wait()
            pltpu.sync_copy(gb1, acc_sh.at[eb.at[r + 3]], add=True)
            return 0
        lax.fori_loop(0, nreal // 2, _pair, 0)
        plsc.subcore_barrier()

        def _wr_full():
            pltpu.sync_copy(acc_sh.at[pl.ds(base, RPS)],
                            acc_out.at[c, pl.ds(base, RPS)])

        def _wr_last():
            pltpu.sync_copy(acc_sh.at[pl.ds((NS - 1) * RPS, RLAST)],
                            acc_out.at[c, pl.ds((NS - 1) * RPS, RLAST)])
        pl.when(s < NS - 1)(_wr_full)
        pl.when(s == NS - 1)(_wr_last)

    return pl.kernel(
        body,
        out_type=[jax.ShapeDtypeStruct((NC, N, D), jnp.float32)],
        mesh=plsc.VectorSubcoreMesh(**_MESH),
        scratch_types=[
            pltpu.VMEM((8, CHUNK), jnp.int32),
            pltpu.VMEM((CHUNK, D), jnp.float32),
            pltpu.VMEM((CHUNK, D), jnp.float32),
            pltpu.VMEM_SHARED((N, D), jnp.float32),
            pltpu.SemaphoreType.DMA,
            pltpu.SemaphoreType.DMA,
        ],
        compiler_params=_NOLAYOUT,
        name="sage_sc_scatter")


def _make_sc_deg():
    def body(dst_hbm, z_hbm, deg_out, dst_v, ones_v, deg_sh):
        c = lax.axis_index("c")
        s = lax.axis_index("s")
        w = s * NC + c
        base = s * RPS

        def _fill(i, _):
            for k in range(D // 16):
                ones_v[i, pl.ds(k * 16, 16)] = jnp.ones((16,), jnp.float32)
            return 0
        lax.fori_loop(0, CHUNK, _fill, 0)

        def _zinit(i, _):
            pltpu.sync_copy(z_hbm, deg_sh.at[pl.ds(base + i * 8, 8)])
            return 0
        lax.fori_loop(0, RPS // 8, _zinit, 0)
        plsc.subcore_barrier()

        pltpu.sync_copy(dst_hbm.at[w], dst_v)
        nreal = jnp.clip(NCHUNKS - w * CPW, 0, CPW)

        def _chunk(j, _):
            pltpu.sync_copy(ones_v, deg_sh.at[dst_v.at[j]], add=True)
            return 0
        lax.fori_loop(0, nreal, _chunk, 0)
        plsc.subcore_barrier()

        pltpu.sync_copy(deg_sh.at[pl.ds(base, RPS)],
                        deg_out.at[c, pl.ds(base, RPS)])

    return pl.kernel(
        body,
        out_type=[jax.ShapeDtypeStruct((NC, R, D), jnp.float32)],
        mesh=plsc.VectorSubcoreMesh(**_MESH),
        scratch_types=[
            pltpu.VMEM((CPW, CHUNK), jnp.int32),
            pltpu.VMEM((CHUNK, D), jnp.float32),
            pltpu.VMEM_SHARED((R, D), jnp.float32),
        ],
        compiler_params=_NOLAYOUT,
        name="sage_sc_deg")


_sc_scatter = _make_sc_scatter()
_sc_deg = _make_sc_deg()


def _mm_body(x_ref, w_ref, o_ref):
    o_ref[...] = jnp.dot(x_ref[...], w_ref[...],
                         preferred_element_type=jnp.float32,
                         precision=lax.Precision.HIGHEST)


def _tc_matmul(x, w):
    return pl.pallas_call(
        _mm_body,
        out_shape=jax.ShapeDtypeStruct((x.shape[0], w.shape[1]), jnp.float32),
    )(x, w)


def _mid_body(p_ref, y_ref, dp_ref, b_ref, w_ref, y2_ref, inv_ref):
    dp = dp_ref[...]
    deg = dp[0] + dp[1]
    inv = 1.0 / (deg + 1.0)
    p = p_ref[...]
    h1 = (p[0, :N] + p[1, :N] + y_ref[...]) * inv + b_ref[...]
    h1 = jnp.maximum(h1, 0.0)
    y2_ref[...] = jnp.dot(h1, w_ref[...], preferred_element_type=jnp.float32,
                          precision=lax.Precision.HIGHEST)
    inv_ref[...] = inv


def _tc_mid(p, y1, dp, b1, W2):
    return pl.pallas_call(
        _mid_body,
        out_shape=[jax.ShapeDtypeStruct((N, D), jnp.float32),
                   jax.ShapeDtypeStruct((N, 1), jnp.float32)],
    )(p, y1, dp, b1, W2)


def _final_body(q_ref, y2_ref, inv_ref, b_ref, o_ref):
    q = q_ref[...]
    o_ref[...] = ((q[0, :N] + q[1, :N] + y2_ref[...]) * inv_ref[...]
                  + b_ref[...])


def _tc_final(q, y2, inv, b2):
    return pl.pallas_call(
        _final_body,
        out_shape=jax.ShapeDtypeStruct((N, D), jnp.float32),
    )(q, y2, inv, b2)


@jax.jit
def kernel(x, edge_index, W1, b1, W2, b2, Wfc, bfc):
    del Wfc, bfc
    src = edge_index[0]
    dst = edge_index[1]
    npad = EPAD - E
    src_p = jnp.concatenate([src, jnp.zeros((npad,), jnp.int32)])
    dst_p = jnp.concatenate([dst, jnp.zeros((npad,), jnp.int32)])
    srcw = src_p.reshape(NW, CPW, CHUNK)
    dstw = dst_p.reshape(NW, CPW, CHUNK)
    ev = jnp.stack([srcw, dstw], axis=2).reshape(NW, 2 * CPW, CHUNK)
    zrows = jnp.zeros((8, D), jnp.float32)

    y1 = _tc_matmul(x, W1)
    (dp,) = _sc_deg(dstw, zrows)
    dcol = dp[:, :N, 0:1]
    (p,) = _sc_scatter(y1, ev, zrows)
    y2, inv = _tc_mid(p, y1, dcol, b1.reshape(1, D), W2)
    (q,) = _sc_scatter(y2, ev, zrows)
    return _tc_final(q, y2, inv, b2.reshape(1, D))


# speedup vs baseline: 1.5827x; 1.5827x over previous
import jax
import jax.numpy as jnp
from jax import lax
from jax.experimental import pallas as pl
from jax.experimental.pallas import tpu as pltpu
from jax.experimental.pallas import tpu_sc as plsc

N = 10000
E = 320000
D = 128
NC = 2
NS = 16
NW = NC * NS
CHUNK = 128
NCHUNKS = E // CHUNK
CPW = 80
EPAD = NW * CPW * CHUNK
R = 10112
RPS = R // NS
RLAST = N - 15 * RPS
assert NCHUNKS % 2 == 0 and CPW % 4 == 0 and RLAST % 8 == 0

_MESH = dict(core_axis_name="c", subcore_axis_name="s",
             num_cores=NC, num_subcores=NS)
_NOLAYOUT = pltpu.CompilerParams(needs_layout_passes=False)


def _make_sc_scatter():
    def body(y_hbm, ev_hbm, z_hbm, acc_out,
             eb, gb0, gb1, acc_sh, sem0, sem1):
        c = lax.axis_index("c")
        s = lax.axis_index("s")
        w = s * NC + c
        base = s * RPS

        def _z_full():
            pltpu.sync_copy(z_hbm, acc_sh.at[pl.ds(base, RPS)])

        def _z_last():
            pltpu.sync_copy(z_hbm.at[pl.ds(0, RLAST)],
                            acc_sh.at[pl.ds(base, RLAST)])
        pl.when(s < NS - 1)(_z_full)
        pl.when(s == NS - 1)(_z_last)
        plsc.subcore_barrier()

        nreal = jnp.clip(NCHUNKS - w * CPW, 0, CPW)

        def _pair(i, _):
            j0 = 2 * i
            g = j0 // 4
            r = 2 * (j0 % 4)

            def _load_group():
                pltpu.sync_copy(ev_hbm.at[w, pl.ds(g * 8, 8)], eb)
            pl.when(j0 % 4 == 0)(_load_group)

            pltpu.async_copy(y_hbm.at[eb.at[r]], gb0, sem0)
            pltpu.async_copy(y_hbm.at[eb.at[r + 2]], gb1, sem1)
            pltpu.make_async_copy(y_hbm.at[eb.at[r]], gb0, sem0).wait()
            pltpu.sync_copy(gb0, acc_sh.at[eb.at[r + 1]], add=True)
            pltpu.make_async_copy(y_hbm.at[eb.at[r + 2]], gb1, sem1).wait()
            pltpu.sync_copy(gb1, acc_sh.at[eb.at[r + 3]], add=True)
            return 0
        lax.fori_loop(0, nreal // 2, _pair, 0)
        plsc.subcore_barrier()

        def _wr_full():
            pltpu.sync_copy(acc_sh.at[pl.ds(base, RPS)],
                            acc_out.at[c, pl.ds(base, RPS)])

        def _wr_last():
            pltpu.sync_copy(acc_sh.at[pl.ds((NS - 1) * RPS, RLAST)],
                            acc_out.at[c, pl.ds((NS - 1) * RPS, RLAST)])
        pl.when(s < NS - 1)(_wr_full)
        pl.when(s == NS - 1)(_wr_last)

    return pl.kernel(
        body,
        out_type=[jax.ShapeDtypeStruct((NC, N, D), jnp.float32)],
        mesh=plsc.VectorSubcoreMesh(**_MESH),
        scratch_types=[
            pltpu.VMEM((8, CHUNK), jnp.int32),
            pltpu.VMEM((CHUNK, D), jnp.float32),
            pltpu.VMEM((CHUNK, D), jnp.float32),
            pltpu.VMEM_SHARED((N, D), jnp.float32),
            pltpu.SemaphoreType.DMA,
            pltpu.SemaphoreType.DMA,
        ],
        compiler_params=_NOLAYOUT,
        name="sage_sc_scatter")


def _make_sc_deg():
    def body(dst_hbm, z_hbm, deg_out, dst_v, ones_v, deg_sh):
        c = lax.axis_index("c")
        s = lax.axis_index("s")
        w = s * NC + c
        base = s * RPS

        def _fill(i, _):
            for k in range(D // 16):
                ones_v[i, pl.ds(k * 16, 16)] = jnp.ones((16,), jnp.float32)
            return 0
        lax.fori_loop(0, CHUNK, _fill, 0)
        pltpu.sync_copy(z_hbm, deg_sh.at[pl.ds(base, RPS)])
        plsc.subcore_barrier()

        pltpu.sync_copy(dst_hbm.at[w], dst_v)
        nreal = jnp.clip(NCHUNKS - w * CPW, 0, CPW)

        def _chunk(j, _):
            pltpu.sync_copy(ones_v, deg_sh.at[dst_v.at[j]], add=True)
            return 0
        lax.fori_loop(0, nreal, _chunk, 0)
        plsc.subcore_barrier()

        pltpu.sync_copy(deg_sh.at[pl.ds(base, RPS)],
                        deg_out.at[c, pl.ds(base, RPS)])

    return pl.kernel(
        body,
        out_type=[jax.ShapeDtypeStruct((NC, R, D), jnp.float32)],
        mesh=plsc.VectorSubcoreMesh(**_MESH),
        scratch_types=[
            pltpu.VMEM((CPW, CHUNK), jnp.int32),
            pltpu.VMEM((CHUNK, D), jnp.float32),
            pltpu.VMEM_SHARED((R, D), jnp.float32),
        ],
        compiler_params=_NOLAYOUT,
        name="sage_sc_deg")


_sc_scatter = _make_sc_scatter()
_sc_deg = _make_sc_deg()


def _mm_body(x_ref, w_ref, o_ref):
    o_ref[...] = jnp.dot(x_ref[...], w_ref[...],
                         preferred_element_type=jnp.float32,
                         precision=lax.Precision.HIGHEST)


def _tc_matmul(x, w):
    return pl.pallas_call(
        _mm_body,
        out_shape=jax.ShapeDtypeStruct((x.shape[0], w.shape[1]), jnp.float32),
    )(x, w)


def _mid_body(p_ref, y_ref, dp_ref, b_ref, w_ref, y2_ref, inv_ref):
    dp = dp_ref[...]
    deg = dp[0] + dp[1]
    inv = 1.0 / (deg + 1.0)
    p = p_ref[...]
    h1 = (p[0, :N] + p[1, :N] + y_ref[...]) * inv + b_ref[...]
    h1 = jnp.maximum(h1, 0.0)
    y2_ref[...] = jnp.dot(h1, w_ref[...], preferred_element_type=jnp.float32,
                          precision=lax.Precision.HIGHEST)
    inv_ref[...] = inv


def _tc_mid(p, y1, dp, b1, W2):
    return pl.pallas_call(
        _mid_body,
        out_shape=[jax.ShapeDtypeStruct((N, D), jnp.float32),
                   jax.ShapeDtypeStruct((N, 1), jnp.float32)],
    )(p, y1, dp, b1, W2)


def _final_body(q_ref, y2_ref, inv_ref, b_ref, o_ref):
    q = q_ref[...]
    o_ref[...] = ((q[0, :N] + q[1, :N] + y2_ref[...]) * inv_ref[...]
                  + b_ref[...])


def _tc_final(q, y2, inv, b2):
    return pl.pallas_call(
        _final_body,
        out_shape=jax.ShapeDtypeStruct((N, D), jnp.float32),
    )(q, y2, inv, b2)


@jax.jit
def kernel(x, edge_index, W1, b1, W2, b2, Wfc, bfc):
    del Wfc, bfc
    src = edge_index[0]
    dst = edge_index[1]
    npad = EPAD - E
    src_p = jnp.concatenate([src, jnp.zeros((npad,), jnp.int32)])
    dst_p = jnp.concatenate([dst, jnp.zeros((npad,), jnp.int32)])
    srcw = src_p.reshape(NW, CPW, CHUNK)
    dstw = dst_p.reshape(NW, CPW, CHUNK)
    ev = jnp.stack([srcw, dstw], axis=2).reshape(NW, 2 * CPW, CHUNK)
    zrows = jnp.zeros((RPS, D), jnp.float32)

    y1 = _tc_matmul(x, W1)
    (dp,) = _sc_deg(dstw, zrows)
    dcol = dp[:, :N, 0:1]
    (p,) = _sc_scatter(y1, ev, zrows)
    y2, inv = _tc_mid(p, y1, dcol, b1.reshape(1, D), W2)
    (q,) = _sc_scatter(y2, ev, zrows)
    return _tc_final(q, y2, inv, b2.reshape(1, D))


# speedup vs baseline: 1.5969x; 1.0090x over previous
import jax
import jax.numpy as jnp
from jax import lax
from jax.experimental import pallas as pl
from jax.experimental.pallas import tpu as pltpu
from jax.experimental.pallas import tpu_sc as plsc

N = 10000
E = 320000
D = 128
NC = 2
NS = 16
NW = NC * NS
CHUNK = 128
NCHUNKS = E // CHUNK
CPW = 80
EPAD = NW * CPW * CHUNK
R = 10112
RPS = R // NS
RLAST = N - 15 * RPS
assert NCHUNKS % 2 == 0 and CPW % 4 == 0 and RLAST % 8 == 0

_MESH = dict(core_axis_name="c", subcore_axis_name="s",
             num_cores=NC, num_subcores=NS)
_NOLAYOUT = pltpu.CompilerParams(needs_layout_passes=False)


def _make_sc_scatter():
    def body(y_hbm, ev_hbm, z_hbm, acc_out,
             eb, gb0, gb1, acc_sh, sem0, sem1, sems0, sems1):
        c = lax.axis_index("c")
        s = lax.axis_index("s")
        w = s * NC + c
        base = s * RPS

        def _z_full():
            pltpu.sync_copy(z_hbm, acc_sh.at[pl.ds(base, RPS)])

        def _z_last():
            pltpu.sync_copy(z_hbm.at[pl.ds(0, RLAST)],
                            acc_sh.at[pl.ds(base, RLAST)])
        pl.when(s < NS - 1)(_z_full)
        pl.when(s == NS - 1)(_z_last)
        plsc.subcore_barrier()

        nreal = jnp.clip(NCHUNKS - w * CPW, 0, CPW)

        def _pair(i, _):
            j0 = 2 * i
            g = j0 // 4
            r = 2 * (j0 % 4)

            def _load_group():
                pltpu.sync_copy(ev_hbm.at[w, pl.ds(g * 8, 8)], eb)
            pl.when(j0 % 4 == 0)(_load_group)

            pltpu.async_copy(y_hbm.at[eb.at[r]], gb0, sem0)
            pltpu.async_copy(y_hbm.at[eb.at[r + 2]], gb1, sem1)
            pltpu.make_async_copy(y_hbm.at[eb.at[r]], gb0, sem0).wait()
            pltpu.async_copy(gb0, acc_sh.at[eb.at[r + 1]], sems0, add=True)
            pltpu.make_async_copy(y_hbm.at[eb.at[r + 2]], gb1, sem1).wait()
            pltpu.async_copy(gb1, acc_sh.at[eb.at[r + 3]], sems1, add=True)
            pltpu.make_async_copy(gb0, acc_sh.at[eb.at[r + 1]], sems0).wait()
            pltpu.make_async_copy(gb1, acc_sh.at[eb.at[r + 3]], sems1).wait()
            return 0
        lax.fori_loop(0, nreal // 2, _pair, 0)
        plsc.subcore_barrier()

        def _wr_full():
            pltpu.sync_copy(acc_sh.at[pl.ds(base, RPS)],
                            acc_out.at[c, pl.ds(base, RPS)])

        def _wr_last():
            pltpu.sync_copy(acc_sh.at[pl.ds((NS - 1) * RPS, RLAST)],
                            acc_out.at[c, pl.ds((NS - 1) * RPS, RLAST)])
        pl.when(s < NS - 1)(_wr_full)
        pl.when(s == NS - 1)(_wr_last)

    return pl.kernel(
        body,
        out_type=[jax.ShapeDtypeStruct((NC, N, D), jnp.float32)],
        mesh=plsc.VectorSubcoreMesh(**_MESH),
        scratch_types=[
            pltpu.VMEM((8, CHUNK), jnp.int32),
            pltpu.VMEM((CHUNK, D), jnp.float32),
            pltpu.VMEM((CHUNK, D), jnp.float32),
            pltpu.VMEM_SHARED((N, D), jnp.float32),
            pltpu.SemaphoreType.DMA,
            pltpu.SemaphoreType.DMA,
            pltpu.SemaphoreType.DMA,
            pltpu.SemaphoreType.DMA,
        ],
        compiler_params=_NOLAYOUT,
        name="sage_sc_scatter")


def _make_sc_deg():
    def body(dst_hbm, z_hbm, deg_out, dst_v, ones_v, deg_sh):
        c = lax.axis_index("c")
        s = lax.axis_index("s")
        w = s * NC + c
        base = s * RPS

        def _fill(i, _):
            for k in range(D // 16):
                ones_v[i, pl.ds(k * 16, 16)] = jnp.ones((16,), jnp.float32)
            return 0
        lax.fori_loop(0, CHUNK, _fill, 0)
        pltpu.sync_copy(z_hbm, deg_sh.at[pl.ds(base, RPS)])
        plsc.subcore_barrier()

        pltpu.sync_copy(dst_hbm.at[w], dst_v)
        nreal = jnp.clip(NCHUNKS - w * CPW, 0, CPW)

        def _chunk(j, _):
            pltpu.sync_copy(ones_v, deg_sh.at[dst_v.at[j]], add=True)
            return 0
        lax.fori_loop(0, nreal, _chunk, 0)
        plsc.subcore_barrier()

        pltpu.sync_copy(deg_sh.at[pl.ds(base, RPS)],
                        deg_out.at[c, pl.ds(base, RPS)])

    return pl.kernel(
        body,
        out_type=[jax.ShapeDtypeStruct((NC, R, D), jnp.float32)],
        mesh=plsc.VectorSubcoreMesh(**_MESH),
        scratch_types=[
            pltpu.VMEM((CPW, CHUNK), jnp.int32),
            pltpu.VMEM((CHUNK, D), jnp.float32),
            pltpu.VMEM_SHARED((R, D), jnp.float32),
        ],
        compiler_params=_NOLAYOUT,
        name="sage_sc_deg")


_sc_scatter = _make_sc_scatter()
_sc_deg = _make_sc_deg()


def _mm_body(x_ref, w_ref, o_ref):
    o_ref[...] = jnp.dot(x_ref[...], w_ref[...],
                         preferred_element_type=jnp.float32,
                         precision=lax.Precision.HIGHEST)


def _tc_matmul(x, w):
    return pl.pallas_call(
        _mm_body,
        out_shape=jax.ShapeDtypeStruct((x.shape[0], w.shape[1]), jnp.float32),
    )(x, w)


def _mid_body(p_ref, y_ref, dp_ref, b_ref, w_ref, y2_ref, inv_ref):
    dp = dp_ref[...]
    deg = dp[0] + dp[1]
    inv = 1.0 / (deg + 1.0)
    p = p_ref[...]
    h1 = (p[0, :N] + p[1, :N] + y_ref[...]) * inv + b_ref[...]
    h1 = jnp.maximum(h1, 0.0)
    y2_ref[...] = jnp.dot(h1, w_ref[...], preferred_element_type=jnp.float32,
                          precision=lax.Precision.HIGHEST)
    inv_ref[...] = inv


def _tc_mid(p, y1, dp, b1, W2):
    return pl.pallas_call(
        _mid_body,
        out_shape=[jax.ShapeDtypeStruct((N, D), jnp.float32),
                   jax.ShapeDtypeStruct((N, 1), jnp.float32)],
    )(p, y1, dp, b1, W2)


def _final_body(q_ref, y2_ref, inv_ref, b_ref, o_ref):
    q = q_ref[...]
    o_ref[...] = ((q[0, :N] + q[1, :N] + y2_ref[...]) * inv_ref[...]
                  + b_ref[...])


def _tc_final(q, y2, inv, b2):
    return pl.pallas_call(
        _final_body,
        out_shape=jax.ShapeDtypeStruct((N, D), jnp.float32),
    )(q, y2, inv, b2)


@jax.jit
def kernel(x, edge_index, W1, b1, W2, b2, Wfc, bfc):
    del Wfc, bfc
    src = edge_index[0]
    dst = edge_index[1]
    npad = EPAD - E
    src_p = jnp.concatenate([src, jnp.zeros((npad,), jnp.int32)])
    dst_p = jnp.concatenate([dst, jnp.zeros((npad,), jnp.int32)])
    srcw = src_p.reshape(NW, CPW, CHUNK)
    dstw = dst_p.reshape(NW, CPW, CHUNK)
    ev = jnp.stack([srcw, dstw], axis=2).reshape(NW, 2 * CPW, CHUNK)
    zrows = jnp.zeros((RPS, D), jnp.float32)

    y1 = _tc_matmul(x, W1)
    (dp,) = _sc_deg(dstw, zrows)
    dcol = dp[:, :N, 0:1]
    (p,) = _sc_scatter(y1, ev, zrows)
    y2, inv = _tc_mid(p, y1, dcol, b1.reshape(1, D), W2)
    (q,) = _sc_scatter(y2, ev, zrows)
    return _tc_final(q, y2, inv, b2.reshape(1, D))
